# Initial kernel scaffold; baseline (speedup 1.0000x reference)
import jax
import jax.numpy as jnp
from jax.experimental import pallas as pl


def kernel(x, edge_index, W1, b1, W2, b2, W3, b3, Wp, bp, Wv, bv):
    raise NotImplementedError("write your pallas kernel here")


# speedup vs baseline: 32.3518x; 32.3518x over previous
import functools

import jax
import jax.numpy as jnp
from jax import lax
from jax.experimental import pallas as pl
from jax.experimental.pallas import tpu as pltpu
from jax.experimental.pallas import tpu_sc as plsc

N = 10000
E = 320000
D_IN = 128
H = 8

NC = 2
NS = 16
NW = NC * NS

C = 128
K = (E + NW * C - 1) // (NW * C)
EPAD = NW * K * C

ACC_N = 10240
ZR = ACC_N // NS

_mesh = plsc.VectorSubcoreMesh(core_axis_name="c", subcore_axis_name="s")
_sc_params = pltpu.CompilerParams(use_tc_tiling_on_sc=False)


def _deg_body(dstw, z1, out, dst_v, ones_v, acc):
    c = lax.axis_index("c")
    s = lax.axis_index("s")
    w = s * NC + c
    pltpu.sync_copy(z1, acc.at[pl.ds(s * ZR, ZR)])
    pltpu.sync_copy(dstw.at[w], dst_v)
    for i in range(C // 16):
        ones_v[pl.ds(i * 16, 16)] = jnp.full((16,), 1.0, jnp.float32)
    plsc.subcore_barrier()

    def step(j, carry):
        pltpu.sync_copy(ones_v, acc.at[dst_v.at[j]], add=True)
        return carry

    lax.fori_loop(0, K, step, 0)
    plsc.subcore_barrier()
    pltpu.sync_copy(acc.at[pl.ds(s * ZR, ZR)], out.at[c, pl.ds(s * ZR, ZR)])


def _sc_degree(dstw, z1):
    return pl.kernel(
        _deg_body,
        out_type=jax.ShapeDtypeStruct((NC, ACC_N), jnp.float32),
        mesh=_mesh,
        scratch_types=[
            pltpu.VMEM((K, C), jnp.int32),
            pltpu.VMEM((C,), jnp.float32),
            pltpu.VMEM_SHARED((ACC_N,), jnp.float32),
        ],
        compiler_params=_sc_params,
    )(dstw, z1)


def _scat_body(g, srcw, dstw, z8, out, src_v, dst_v, rows_v, acc, sem):
    c = lax.axis_index("c")
    s = lax.axis_index("s")
    w = s * NC + c
    pltpu.sync_copy(z8, acc.at[pl.ds(s * ZR, ZR)])
    pltpu.sync_copy(srcw.at[w], src_v)
    pltpu.sync_copy(dstw.at[w], dst_v)
    plsc.subcore_barrier()

    def step(j, carry):
        pltpu.async_copy(g.at[src_v.at[j]], rows_v, sem).wait()
        pltpu.sync_copy(rows_v, acc.at[dst_v.at[j]], add=True)
        return carry

    lax.fori_loop(0, K, step, 0)
    plsc.subcore_barrier()
    pltpu.sync_copy(acc.at[pl.ds(s * ZR, ZR)], out.at[c, pl.ds(s * ZR, ZR)])


def _sc_scatter(g, srcw, dstw, z8):
    return pl.kernel(
        _scat_body,
        out_type=jax.ShapeDtypeStruct((NC, ACC_N, H), jnp.float32),
        mesh=_mesh,
        scratch_types=[
            pltpu.VMEM((K, C), jnp.int32),
            pltpu.VMEM((K, C), jnp.int32),
            pltpu.VMEM((C, H), jnp.float32),
            pltpu.VMEM_SHARED((ACC_N, H), jnp.float32),
            pltpu.SemaphoreType.DMA,
        ],
        compiler_params=_sc_params,
    )(g, srcw, dstw, z8)


def _prep_body(x_ref, w1_ref, degt_ref, g_ref, dinv_ref):
    deg = degt_ref[:N, 0:1] + degt_ref[:N, 1:2] + 1.0
    dinv = lax.rsqrt(deg)
    h = jnp.dot(x_ref[...], w1_ref[...], preferred_element_type=jnp.float32)
    dinv_ref[...] = dinv
    g_ref[...] = h * dinv


def _tc_prep(x, W1, degt):
    return pl.pallas_call(
        _prep_body,
        out_shape=(
            jax.ShapeDtypeStruct((N, H), jnp.float32),
            jax.ShapeDtypeStruct((N, 1), jnp.float32),
        ),
    )(x, W1, degt)


def _layer_body(accp_ref, g_ref, dinv_ref, w_ref, b_ref, out_ref):
    acc = accp_ref[0, :N, :] + accp_ref[1, :N, :] + g_ref[...]
    hh = jnp.maximum(acc * dinv_ref[...] + b_ref[...], 0.0)
    out_ref[...] = jnp.dot(hh, w_ref[...],
                           preferred_element_type=jnp.float32) * dinv_ref[...]


def _tc_layer(accp, g_prev, dinv, W, b):
    return pl.pallas_call(
        _layer_body,
        out_shape=jax.ShapeDtypeStruct((N, H), jnp.float32),
    )(accp, g_prev, dinv, W, b)


def _final_body(accp_ref, g_ref, dinv_ref, b_ref, wp_ref, bp_ref, wv_ref,
                bv_ref, proba_ref, value_ref):
    acc = accp_ref[0, :N, :] + accp_ref[1, :N, :] + g_ref[...]
    h3 = jnp.maximum(acc * dinv_ref[...] + b_ref[...], 0.0)
    proba_ref[...] = jnp.dot(h3, wp_ref[...],
                             preferred_element_type=jnp.float32) + bp_ref[...]
    mean = jnp.mean(h3, axis=0, keepdims=True)
    value_ref[...] = jnp.dot(mean, wv_ref[...],
                             preferred_element_type=jnp.float32) + bv_ref[...]


def _tc_final(accp, g3, dinv, b3, Wp, bp, Wv, bv):
    return pl.pallas_call(
        _final_body,
        out_shape=(
            jax.ShapeDtypeStruct((N, 1), jnp.float32),
            jax.ShapeDtypeStruct((1, 1), jnp.float32),
        ),
    )(accp, g3, dinv, b3, Wp, bp, Wv, bv)


def kernel(x, edge_index, W1, b1, W2, b2, W3, b3, Wp, bp, Wv, bv):
    src = edge_index[0].astype(jnp.int32)
    dst = edge_index[1].astype(jnp.int32)
    pad_i = jnp.arange(EPAD - E, dtype=jnp.int32)
    src_p = jnp.concatenate([src, pad_i % N])
    dst_p = jnp.concatenate([dst, N + pad_i % (ACC_N - N)])
    srcw = src_p.reshape(NW, K, C)
    dstw = dst_p.reshape(NW, K, C)

    z1 = jnp.zeros((ZR,), jnp.float32)
    z8 = jnp.zeros((ZR, H), jnp.float32)

    degp = _sc_degree(dstw, z1)
    degt = degp.T
    g1, dinv = _tc_prep(x, W1, degt)

    acc1 = _sc_scatter(g1, srcw, dstw, z8)
    g2 = _tc_layer(acc1, g1, dinv, W2, b1.reshape(1, H))
    acc2 = _sc_scatter(g2, srcw, dstw, z8)
    g3 = _tc_layer(acc2, g2, dinv, W3, b2.reshape(1, H))
    acc3 = _sc_scatter(g3, srcw, dstw, z8)
    proba, value = _tc_final(acc3, g3, dinv, b3.reshape(1, H),
                             Wp, bp.reshape(1, 1), Wv, bv.reshape(1, 1))
    return (proba, value)


# speedup vs baseline: 58.8630x; 1.8195x over previous
import functools

import jax
import jax.numpy as jnp
from jax import lax
from jax.experimental import pallas as pl
from jax.experimental.pallas import tpu as pltpu
from jax.experimental.pallas import tpu_sc as plsc

N = 10000
E = 320000
D_IN = 128
H = 8

NC = 2
NS = 16
NW = NC * NS

C = 1024
K = (E + NW * C - 1) // (NW * C)
EPAD = NW * K * C

ACC_N = 10240
ZR = ACC_N // NS

_mesh = plsc.VectorSubcoreMesh(core_axis_name="c", subcore_axis_name="s")
_sc_params = pltpu.CompilerParams(use_tc_tiling_on_sc=False)


def _deg_body(dstw, z1, out, dst_v, ones_v, acc, sems):
    c = lax.axis_index("c")
    s = lax.axis_index("s")
    w = s * NC + c
    pltpu.sync_copy(z1, acc.at[pl.ds(s * ZR, ZR)])
    pltpu.sync_copy(dstw.at[w], dst_v)
    for i in range(C // 16):
        ones_v[pl.ds(i * 16, 16)] = jnp.full((16,), 1.0, jnp.float32)
    plsc.subcore_barrier()

    descs = [
        pltpu.async_copy(ones_v, acc.at[dst_v.at[j]], sems.at[j], add=True)
        for j in range(K)
    ]
    for d in descs:
        d.wait()
    plsc.subcore_barrier()
    pltpu.sync_copy(acc.at[pl.ds(s * ZR, ZR)], out.at[c, pl.ds(s * ZR, ZR)])


def _sc_degree(dstw, z1):
    return pl.kernel(
        _deg_body,
        out_type=jax.ShapeDtypeStruct((NC, ACC_N), jnp.float32),
        mesh=_mesh,
        scratch_types=[
            pltpu.VMEM((K, C), jnp.int32),
            pltpu.VMEM((C,), jnp.float32),
            pltpu.VMEM_SHARED((ACC_N,), jnp.float32),
            pltpu.SemaphoreType.DMA((K,)),
        ],
        compiler_params=_sc_params,
    )(dstw, z1)


def _scat_body(g, srcw, dstw, z8, out, src_v, dst_v, rows_v, acc, semg, sems):
    c = lax.axis_index("c")
    s = lax.axis_index("s")
    w = s * NC + c
    pltpu.sync_copy(z8, acc.at[pl.ds(s * ZR, ZR)])
    pltpu.sync_copy(srcw.at[w], src_v)
    pltpu.sync_copy(dstw.at[w], dst_v)
    plsc.subcore_barrier()

    gds = [
        pltpu.async_copy(g.at[src_v.at[j]], rows_v.at[j], semg.at[j])
        for j in range(K)
    ]
    sds = []
    for j in range(K):
        gds[j].wait()
        sds.append(pltpu.async_copy(rows_v.at[j], acc.at[dst_v.at[j]],
                                    sems.at[j], add=True))
    for d in sds:
        d.wait()
    plsc.subcore_barrier()
    pltpu.sync_copy(acc.at[pl.ds(s * ZR, ZR)], out.at[c, pl.ds(s * ZR, ZR)])


def _sc_scatter(g, srcw, dstw, z8):
    return pl.kernel(
        _scat_body,
        out_type=jax.ShapeDtypeStruct((NC, ACC_N, H), jnp.float32),
        mesh=_mesh,
        scratch_types=[
            pltpu.VMEM((K, C), jnp.int32),
            pltpu.VMEM((K, C), jnp.int32),
            pltpu.VMEM((K, C, H), jnp.float32),
            pltpu.VMEM_SHARED((ACC_N, H), jnp.float32),
            pltpu.SemaphoreType.DMA((K,)),
            pltpu.SemaphoreType.DMA((K,)),
        ],
        compiler_params=_sc_params,
    )(g, srcw, dstw, z8)


def _prep_body(x_ref, w1_ref, degt_ref, g_ref, dinv_ref):
    deg = degt_ref[:N, 0:1] + degt_ref[:N, 1:2] + 1.0
    dinv = lax.rsqrt(deg)
    h = jnp.dot(x_ref[...], w1_ref[...], preferred_element_type=jnp.float32)
    dinv_ref[...] = dinv
    g_ref[...] = h * dinv


def _tc_prep(x, W1, degt):
    return pl.pallas_call(
        _prep_body,
        out_shape=(
            jax.ShapeDtypeStruct((N, H), jnp.float32),
            jax.ShapeDtypeStruct((N, 1), jnp.float32),
        ),
    )(x, W1, degt)


def _layer_body(accp_ref, g_ref, dinv_ref, w_ref, b_ref, out_ref):
    acc = accp_ref[0, :N, :] + accp_ref[1, :N, :] + g_ref[...]
    hh = jnp.maximum(acc * dinv_ref[...] + b_ref[...], 0.0)
    out_ref[...] = jnp.dot(hh, w_ref[...],
                           preferred_element_type=jnp.float32) * dinv_ref[...]


def _tc_layer(accp, g_prev, dinv, W, b):
    return pl.pallas_call(
        _layer_body,
        out_shape=jax.ShapeDtypeStruct((N, H), jnp.float32),
    )(accp, g_prev, dinv, W, b)


def _final_body(accp_ref, g_ref, dinv_ref, b_ref, wp_ref, bp_ref, wv_ref,
                bv_ref, proba_ref, value_ref):
    acc = accp_ref[0, :N, :] + accp_ref[1, :N, :] + g_ref[...]
    h3 = jnp.maximum(acc * dinv_ref[...] + b_ref[...], 0.0)
    proba_ref[...] = jnp.dot(h3, wp_ref[...],
                             preferred_element_type=jnp.float32) + bp_ref[...]
    mean = jnp.mean(h3, axis=0, keepdims=True)
    value_ref[...] = jnp.dot(mean, wv_ref[...],
                             preferred_element_type=jnp.float32) + bv_ref[...]


def _tc_final(accp, g3, dinv, b3, Wp, bp, Wv, bv):
    return pl.pallas_call(
        _final_body,
        out_shape=(
            jax.ShapeDtypeStruct((N, 1), jnp.float32),
            jax.ShapeDtypeStruct((1, 1), jnp.float32),
        ),
    )(accp, g3, dinv, b3, Wp, bp, Wv, bv)


def kernel(x, edge_index, W1, b1, W2, b2, W3, b3, Wp, bp, Wv, bv):
    src = edge_index[0].astype(jnp.int32)
    dst = edge_index[1].astype(jnp.int32)
    pad_i = jnp.arange(EPAD - E, dtype=jnp.int32)
    src_p = jnp.concatenate([src, pad_i % N])
    dst_p = jnp.concatenate([dst, N + pad_i % (ACC_N - N)])
    srcw = src_p.reshape(NW, K, C)
    dstw = dst_p.reshape(NW, K, C)

    z1 = jnp.zeros((ZR,), jnp.float32)
    z8 = jnp.zeros((ZR, H), jnp.float32)

    degp = _sc_degree(dstw, z1)
    degt = degp.T
    g1, dinv = _tc_prep(x, W1, degt)

    acc1 = _sc_scatter(g1, srcw, dstw, z8)
    g2 = _tc_layer(acc1, g1, dinv, W2, b1.reshape(1, H))
    acc2 = _sc_scatter(g2, srcw, dstw, z8)
    g3 = _tc_layer(acc2, g2, dinv, W3, b2.reshape(1, H))
    acc3 = _sc_scatter(g3, srcw, dstw, z8)
    proba, value = _tc_final(acc3, g3, dinv, b3.reshape(1, H),
                             Wp, bp.reshape(1, 1), Wv, bv.reshape(1, 1))
    return (proba, value)


# speedup vs baseline: 61.2354x; 1.0403x over previous
import jax
import jax.numpy as jnp
from jax import lax
from jax.experimental import pallas as pl
from jax.experimental.pallas import tpu as pltpu
from jax.experimental.pallas import tpu_sc as plsc

N = 10000
E = 320000
D_IN = 128
H = 8

NC = 2
NS = 16
NW = NC * NS

C = 2000
K = E // (NW * C)

ACC_N = 10240
ZR = ACC_N // NS

_mesh = plsc.VectorSubcoreMesh(core_axis_name="c", subcore_axis_name="s")
_sc_params = pltpu.CompilerParams(use_tc_tiling_on_sc=False)


def _deg_body(dstw, ones_h, z1, out, dst_v, ones_v, acc, semp, sems):
    c = lax.axis_index("c")
    s = lax.axis_index("s")
    w = s * NC + c
    pz = pltpu.async_copy(z1.at[pl.ds(s * ZR, ZR)], acc.at[pl.ds(s * ZR, ZR)],
                          semp.at[0])
    pd = pltpu.async_copy(dstw.at[w], dst_v, semp.at[1])
    po = pltpu.async_copy(ones_h, ones_v, semp.at[2])
    pz.wait()
    pd.wait()
    po.wait()
    plsc.subcore_barrier()

    descs = [
        pltpu.async_copy(ones_v, acc.at[dst_v.at[j]], sems.at[j], add=True)
        for j in range(K)
    ]
    for d in descs:
        d.wait()
    plsc.subcore_barrier()
    pltpu.sync_copy(acc.at[pl.ds(s * ZR, ZR)], out.at[c, pl.ds(s * ZR, ZR)])


def _sc_degree(dstw, ones_h, z1):
    return pl.kernel(
        _deg_body,
        out_type=jax.ShapeDtypeStruct((NC, ACC_N), jnp.float32),
        mesh=_mesh,
        scratch_types=[
            pltpu.VMEM((K, C), jnp.int32),
            pltpu.VMEM((C,), jnp.float32),
            pltpu.VMEM_SHARED((ACC_N,), jnp.float32),
            pltpu.SemaphoreType.DMA((3,)),
            pltpu.SemaphoreType.DMA((K,)),
        ],
        compiler_params=_sc_params,
    )(dstw, ones_h, z1)


def _scat_body(g, srcw, dstw, z8, out, src_v, dst_v, rows_v, acc, semp,
               semg, sems):
    c = lax.axis_index("c")
    s = lax.axis_index("s")
    w = s * NC + c
    pz = pltpu.async_copy(z8.at[pl.ds(s * ZR, ZR)], acc.at[pl.ds(s * ZR, ZR)],
                          semp.at[0])
    ps = pltpu.async_copy(srcw.at[w], src_v, semp.at[1])
    pd = pltpu.async_copy(dstw.at[w], dst_v, semp.at[2])
    ps.wait()
    gds = [
        pltpu.async_copy(g.at[src_v.at[j]], rows_v.at[j], semg.at[j])
        for j in range(K)
    ]
    pz.wait()
    pd.wait()
    plsc.subcore_barrier()
    sds = []
    for j in range(K):
        gds[j].wait()
        sds.append(pltpu.async_copy(rows_v.at[j], acc.at[dst_v.at[j]],
                                    sems.at[j], add=True))
    for d in sds:
        d.wait()
    plsc.subcore_barrier()
    pltpu.sync_copy(acc.at[pl.ds(s * ZR, ZR)], out.at[c, pl.ds(s * ZR, ZR)])


def _sc_scatter(g, srcw, dstw, z8):
    return pl.kernel(
        _scat_body,
        out_type=jax.ShapeDtypeStruct((NC, ACC_N, H), jnp.float32),
        mesh=_mesh,
        scratch_types=[
            pltpu.VMEM((K, C), jnp.int32),
            pltpu.VMEM((K, C), jnp.int32),
            pltpu.VMEM((K, C, H), jnp.float32),
            pltpu.VMEM_SHARED((ACC_N, H), jnp.float32),
            pltpu.SemaphoreType.DMA((3,)),
            pltpu.SemaphoreType.DMA((K,)),
            pltpu.SemaphoreType.DMA((K,)),
        ],
        compiler_params=_sc_params,
    )(g, srcw, dstw, z8)


def _mm1_body(x_ref, w1_ref, h_ref):
    h_ref[...] = jnp.dot(x_ref[...], w1_ref[...],
                         preferred_element_type=jnp.float32)


def _tc_mm1(x, W1):
    return pl.pallas_call(
        _mm1_body,
        out_shape=jax.ShapeDtypeStruct((N, H), jnp.float32),
    )(x, W1)


def _scale_body(h_ref, degt_ref, g_ref, dinv_ref):
    deg = degt_ref[:N, 0:1] + degt_ref[:N, 1:2] + 1.0
    dinv = lax.rsqrt(deg)
    dinv_ref[...] = dinv
    g_ref[...] = h_ref[...] * dinv


def _tc_scale(h1, degt):
    return pl.pallas_call(
        _scale_body,
        out_shape=(
            jax.ShapeDtypeStruct((N, H), jnp.float32),
            jax.ShapeDtypeStruct((N, 1), jnp.float32),
        ),
    )(h1, degt)


def _layer_body(accp_ref, g_ref, dinv_ref, w_ref, b_ref, out_ref):
    acc = accp_ref[0, :N, :] + accp_ref[1, :N, :] + g_ref[...]
    hh = jnp.maximum(acc * dinv_ref[...] + b_ref[...], 0.0)
    out_ref[...] = jnp.dot(hh, w_ref[...],
                           preferred_element_type=jnp.float32) * dinv_ref[...]


def _tc_layer(accp, g_prev, dinv, W, b):
    return pl.pallas_call(
        _layer_body,
        out_shape=jax.ShapeDtypeStruct((N, H), jnp.float32),
    )(accp, g_prev, dinv, W, b)


def _final_body(accp_ref, g_ref, dinv_ref, b_ref, wp_ref, bp_ref, wv_ref,
                bv_ref, proba_ref, value_ref):
    acc = accp_ref[0, :N, :] + accp_ref[1, :N, :] + g_ref[...]
    h3 = jnp.maximum(acc * dinv_ref[...] + b_ref[...], 0.0)
    proba_ref[...] = jnp.dot(h3, wp_ref[...],
                             preferred_element_type=jnp.float32) + bp_ref[...]
    mean = jnp.mean(h3, axis=0, keepdims=True)
    value_ref[...] = jnp.dot(mean, wv_ref[...],
                             preferred_element_type=jnp.float32) + bv_ref[...]


def _tc_final(accp, g3, dinv, b3, Wp, bp, Wv, bv):
    return pl.pallas_call(
        _final_body,
        out_shape=(
            jax.ShapeDtypeStruct((N, 1), jnp.float32),
            jax.ShapeDtypeStruct((1, 1), jnp.float32),
        ),
    )(accp, g3, dinv, b3, Wp, bp, Wv, bv)


def kernel(x, edge_index, W1, b1, W2, b2, W3, b3, Wp, bp, Wv, bv):
    src = edge_index[0].astype(jnp.int32)
    dst = edge_index[1].astype(jnp.int32)
    srcw = src.reshape(NW, K, C)
    dstw = dst.reshape(NW, K, C)

    ones_h = jnp.ones((C,), jnp.float32)
    z1 = jnp.zeros((ACC_N,), jnp.float32)
    z8 = jnp.zeros((ACC_N, H), jnp.float32)

    degp = _sc_degree(dstw, ones_h, z1)
    h1 = _tc_mm1(x, W1)
    g1, dinv = _tc_scale(h1, degp.T)

    acc1 = _sc_scatter(g1, srcw, dstw, z8)
    g2 = _tc_layer(acc1, g1, dinv, W2, b1.reshape(1, H))
    acc2 = _sc_scatter(g2, srcw, dstw, z8)
    g3 = _tc_layer(acc2, g2, dinv, W3, b2.reshape(1, H))
    acc3 = _sc_scatter(g3, srcw, dstw, z8)
    proba, value = _tc_final(acc3, g3, dinv, b3.reshape(1, H),
                             Wp, bp.reshape(1, 1), Wv, bv.reshape(1, 1))
    return (proba, value)


# speedup vs baseline: 64.1623x; 1.0478x over previous
import jax
import jax.numpy as jnp
from jax import lax
from jax.experimental import pallas as pl
from jax.experimental.pallas import tpu as pltpu
from jax.experimental.pallas import tpu_sc as plsc

N = 10000
E = 320000
D_IN = 128
H = 8

NC = 2
NS = 16
NW = NC * NS

C = 2000
K = E // (NW * C)

ACC_N = 10240
ZR = ACC_N // NS

_mesh = plsc.VectorSubcoreMesh(core_axis_name="c", subcore_axis_name="s")
_sc_params = pltpu.CompilerParams(use_tc_tiling_on_sc=False)


def _deg_body(edges, ones_h, z1, out, dst_v, ones_v, acc, semp, sems):
    c = lax.axis_index("c")
    s = lax.axis_index("s")
    w = s * NC + c
    pz = pltpu.async_copy(z1.at[pl.ds(s * ZR, ZR)], acc.at[pl.ds(s * ZR, ZR)],
                          semp.at[0])
    pds = [
        pltpu.async_copy(edges.at[1, pl.ds(w * K * C + j * C, C)],
                         dst_v.at[j], semp.at[1])
        for j in range(K)
    ]
    po = pltpu.async_copy(ones_h, ones_v, semp.at[2])
    pz.wait()
    for p in pds:
        p.wait()
    po.wait()
    plsc.subcore_barrier()

    descs = [
        pltpu.async_copy(ones_v, acc.at[dst_v.at[j]], sems.at[j], add=True)
        for j in range(K)
    ]
    for d in descs:
        d.wait()
    plsc.subcore_barrier()
    pltpu.sync_copy(acc.at[pl.ds(s * ZR, ZR)], out.at[c, pl.ds(s * ZR, ZR)])


def _sc_degree(edges, ones_h, z1):
    return pl.kernel(
        _deg_body,
        out_type=jax.ShapeDtypeStruct((NC, ACC_N), jnp.float32),
        mesh=_mesh,
        scratch_types=[
            pltpu.VMEM((K, C), jnp.int32),
            pltpu.VMEM((C,), jnp.float32),
            pltpu.VMEM_SHARED((ACC_N,), jnp.float32),
            pltpu.SemaphoreType.DMA((3,)),
            pltpu.SemaphoreType.DMA((K,)),
        ],
        compiler_params=_sc_params,
    )(edges, ones_h, z1)


def _scat_body(g, edges, z8, out, src_v, dst_v, rows_v, acc, semp,
               semg, sems):
    c = lax.axis_index("c")
    s = lax.axis_index("s")
    w = s * NC + c
    base = w * K * C
    pz = pltpu.async_copy(z8.at[pl.ds(s * ZR, ZR)], acc.at[pl.ds(s * ZR, ZR)],
                          semp.at[0])
    pss = [
        pltpu.async_copy(edges.at[0, pl.ds(base + j * C, C)], src_v.at[j],
                         semp.at[1])
        for j in range(K)
    ]
    pds = [
        pltpu.async_copy(edges.at[1, pl.ds(base + j * C, C)], dst_v.at[j],
                         semp.at[2])
        for j in range(K)
    ]
    for p in pss:
        p.wait()
    gds = [
        pltpu.async_copy(g.at[src_v.at[j]], rows_v.at[j], semg.at[j])
        for j in range(K)
    ]
    pz.wait()
    for p in pds:
        p.wait()
    plsc.subcore_barrier()
    sds = []
    for j in range(K):
        gds[j].wait()
        sds.append(pltpu.async_copy(rows_v.at[j], acc.at[dst_v.at[j]],
                                    sems.at[j], add=True))
    for d in sds:
        d.wait()
    plsc.subcore_barrier()
    pltpu.sync_copy(acc.at[pl.ds(s * ZR, ZR)], out.at[c, pl.ds(s * ZR, ZR)])


def _sc_scatter(g, edges, z8):
    return pl.kernel(
        _scat_body,
        out_type=jax.ShapeDtypeStruct((NC, ACC_N, H), jnp.float32),
        mesh=_mesh,
        scratch_types=[
            pltpu.VMEM((K, C), jnp.int32),
            pltpu.VMEM((K, C), jnp.int32),
            pltpu.VMEM((K, C, H), jnp.float32),
            pltpu.VMEM_SHARED((ACC_N, H), jnp.float32),
            pltpu.SemaphoreType.DMA((3,)),
            pltpu.SemaphoreType.DMA((K,)),
            pltpu.SemaphoreType.DMA((K,)),
        ],
        compiler_params=_sc_params,
    )(g, edges, z8)


def _mm1_body(x_ref, w1_ref, h_ref):
    h_ref[...] = jnp.dot(x_ref[...], w1_ref[...],
                         preferred_element_type=jnp.float32)


def _tc_mm1(x, W1):
    return pl.pallas_call(
        _mm1_body,
        out_shape=jax.ShapeDtypeStruct((N, H), jnp.float32),
    )(x, W1)


def _scale_body(h_ref, degt_ref, g_ref, dinv_ref):
    deg = degt_ref[:N, 0:1] + degt_ref[:N, 1:2] + 1.0
    dinv = lax.rsqrt(deg)
    dinv_ref[...] = dinv
    g_ref[...] = h_ref[...] * dinv


def _tc_scale(h1, degt):
    return pl.pallas_call(
        _scale_body,
        out_shape=(
            jax.ShapeDtypeStruct((N, H), jnp.float32),
            jax.ShapeDtypeStruct((N, 1), jnp.float32),
        ),
    )(h1, degt)


def _layer_body(accp_ref, g_ref, dinv_ref, w_ref, b_ref, out_ref):
    acc = accp_ref[0, :N, :] + accp_ref[1, :N, :] + g_ref[...]
    hh = jnp.maximum(acc * dinv_ref[...] + b_ref[...], 0.0)
    out_ref[...] = jnp.dot(hh, w_ref[...],
                           preferred_element_type=jnp.float32) * dinv_ref[...]


def _tc_layer(accp, g_prev, dinv, W, b):
    return pl.pallas_call(
        _layer_body,
        out_shape=jax.ShapeDtypeStruct((N, H), jnp.float32),
    )(accp, g_prev, dinv, W, b)


def _final_body(accp_ref, g_ref, dinv_ref, b_ref, wp_ref, bp_ref, wv_ref,
                bv_ref, proba_ref, value_ref):
    acc = accp_ref[0, :N, :] + accp_ref[1, :N, :] + g_ref[...]
    h3 = jnp.maximum(acc * dinv_ref[...] + b_ref[...], 0.0)
    proba_ref[...] = jnp.dot(h3, wp_ref[...],
                             preferred_element_type=jnp.float32) + bp_ref[...]
    mean = jnp.mean(h3, axis=0, keepdims=True)
    value_ref[...] = jnp.dot(mean, wv_ref[...],
                             preferred_element_type=jnp.float32) + bv_ref[...]


def _tc_final(accp, g3, dinv, b3, Wp, bp, Wv, bv):
    return pl.pallas_call(
        _final_body,
        out_shape=(
            jax.ShapeDtypeStruct((N, 1), jnp.float32),
            jax.ShapeDtypeStruct((1, 1), jnp.float32),
        ),
    )(accp, g3, dinv, b3, Wp, bp, Wv, bv)


def kernel(x, edge_index, W1, b1, W2, b2, W3, b3, Wp, bp, Wv, bv):
    edges = edge_index.astype(jnp.int32)

    ones_h = jnp.ones((C,), jnp.float32)
    z1 = jnp.zeros((ACC_N,), jnp.float32)
    z8 = jnp.zeros((ACC_N, H), jnp.float32)

    degp = _sc_degree(edges, ones_h, z1)
    h1 = _tc_mm1(x, W1)
    g1, dinv = _tc_scale(h1, degp.T)

    acc1 = _sc_scatter(g1, edges, z8)
    g2 = _tc_layer(acc1, g1, dinv, W2, b1.reshape(1, H))
    acc2 = _sc_scatter(g2, edges, z8)
    g3 = _tc_layer(acc2, g2, dinv, W3, b2.reshape(1, H))
    acc3 = _sc_scatter(g3, edges, z8)
    proba, value = _tc_final(acc3, g3, dinv, b3.reshape(1, H),
                             Wp, bp.reshape(1, 1), Wv, bv.reshape(1, 1))
    return (proba, value)


# speedup vs baseline: 107.8510x; 1.6809x over previous
import jax
import jax.numpy as jnp
from jax import lax
from jax.experimental import pallas as pl
from jax.experimental.pallas import tpu as pltpu
from jax.experimental.pallas import tpu_sc as plsc

N = 10000
E = 320000
D_IN = 128
H = 8

NC = 2
NS = 16
NW = NC * NS

C = 2000
K = E // (NW * C)

ACC_N = 10240
ZR = ACC_N // NS
FR = ACC_N // 16
NFR = N // 16

_mesh = plsc.VectorSubcoreMesh(core_axis_name="c", subcore_axis_name="s")
_sc_params = pltpu.CompilerParams(use_tc_tiling_on_sc=False)


def _deg_body(edges, ones_h, z1, out, dst_v, ones_v, acc, semp, sems):
    c = lax.axis_index("c")
    s = lax.axis_index("s")
    w = s * NC + c
    pz = pltpu.async_copy(z1.at[pl.ds(s * ZR, ZR)], acc.at[pl.ds(s * ZR, ZR)],
                          semp.at[0])
    pds = [pltpu.async_copy(edges.at[1, pl.ds(w * K * C + j * C, C)],
                            dst_v.at[j], semp.at[1]) for j in range(K)]
    po = pltpu.async_copy(ones_h, ones_v, semp.at[2])
    pz.wait()
    for p in pds:
        p.wait()
    po.wait()
    plsc.subcore_barrier()

    descs = [
        pltpu.async_copy(ones_v, acc.at[dst_v.at[j]], sems.at[j], add=True)
        for j in range(K)
    ]
    for d in descs:
        d.wait()
    plsc.subcore_barrier()
    pltpu.sync_copy(acc.at[pl.ds(s * ZR, ZR)], out.at[c, pl.ds(s * ZR, ZR)])


def _sc_degree(edges, ones_h, z1):
    return pl.kernel(
        _deg_body,
        out_type=jax.ShapeDtypeStruct((NC, ACC_N), jnp.float32),
        mesh=_mesh,
        scratch_types=[
            pltpu.VMEM((K, C), jnp.int32),
            pltpu.VMEM((C,), jnp.float32),
            pltpu.VMEM_SHARED((ACC_N,), jnp.float32),
            pltpu.SemaphoreType.DMA((3,)),
            pltpu.SemaphoreType.DMA((K,)),
        ],
        compiler_params=_sc_params,
    )(edges, ones_h, z1)


def _scat_body(g, edges, z8, out, src_v, dst_v, rows_v, acc, semp,
               semg, sems):
    c = lax.axis_index("c")
    s = lax.axis_index("s")
    w = s * NC + c
    base = w * K * C
    pz = pltpu.async_copy(z8.at[pl.ds(s * ZR, ZR)], acc.at[pl.ds(s * ZR, ZR)],
                          semp.at[0])
    pss = [pltpu.async_copy(edges.at[0, pl.ds(base + j * C, C)], src_v.at[j],
                            semp.at[1]) for j in range(K)]
    pds = [pltpu.async_copy(edges.at[1, pl.ds(base + j * C, C)], dst_v.at[j],
                            semp.at[2]) for j in range(K)]
    for p in pss:
        p.wait()
    gds = [
        pltpu.async_copy(g.at[src_v.at[j]], rows_v.at[j], semg.at[j])
        for j in range(K)
    ]
    pz.wait()
    for p in pds:
        p.wait()
    plsc.subcore_barrier()
    sds = []
    for j in range(K):
        gds[j].wait()
        sds.append(pltpu.async_copy(rows_v.at[j], acc.at[dst_v.at[j]],
                                    sems.at[j], add=True))
    for d in sds:
        d.wait()
    plsc.subcore_barrier()
    pltpu.sync_copy(acc.at[pl.ds(s * ZR, ZR)], out.at[c, pl.ds(s * ZR, ZR)])


def _sc_scatter(g, edges, z8):
    return pl.kernel(
        _scat_body,
        out_type=jax.ShapeDtypeStruct((NC, ACC_N, H), jnp.float32),
        mesh=_mesh,
        scratch_types=[
            pltpu.VMEM((K, C), jnp.int32),
            pltpu.VMEM((K, C), jnp.int32),
            pltpu.VMEM((K, C, H), jnp.float32),
            pltpu.VMEM_SHARED((ACC_N, H), jnp.float32),
            pltpu.SemaphoreType.DMA((3,)),
            pltpu.SemaphoreType.DMA((K,)),
            pltpu.SemaphoreType.DMA((K,)),
        ],
        compiler_params=_sc_params,
    )(g, edges, z8)


def _mm1_body(xg_ref, bw1_ref, h_ref):
    h_ref[...] = jnp.dot(xg_ref[...], bw1_ref[...],
                         preferred_element_type=jnp.float32)


def _tc_mm1(xg, BigW1):
    return pl.pallas_call(
        _mm1_body,
        out_shape=jax.ShapeDtypeStruct((FR, 128), jnp.float32),
    )(xg, BigW1)


def _scale_body(h_ref, degt_ref, brd_ref, g_ref, dinvx_ref):
    deg = degt_ref[0] + degt_ref[1] + 1.0
    dinv = lax.rsqrt(deg)
    dinvx = jnp.dot(dinv, brd_ref[...], preferred_element_type=jnp.float32)
    dinvx_ref[...] = dinvx
    g_ref[...] = h_ref[...] * dinvx


def _tc_scale(h1, degt_rows, Brd):
    return pl.pallas_call(
        _scale_body,
        out_shape=(
            jax.ShapeDtypeStruct((FR, 128), jnp.float32),
            jax.ShapeDtypeStruct((FR, 128), jnp.float32),
        ),
    )(h1, degt_rows, Brd)


def _layer_body(accp_ref, g_ref, dinvx_ref, wbd_ref, b_ref, out_ref):
    dinvx = dinvx_ref[...]
    acc = accp_ref[0] + accp_ref[1] + g_ref[...]
    hh = jnp.maximum(acc * dinvx + b_ref[...], 0.0)
    out_ref[...] = jnp.dot(hh, wbd_ref[...],
                           preferred_element_type=jnp.float32) * dinvx


def _tc_layer(accpf, g_prev, dinvx, WBD, bf):
    return pl.pallas_call(
        _layer_body,
        out_shape=jax.ShapeDtypeStruct((FR, 128), jnp.float32),
    )(accpf, g_prev, dinvx, WBD, bf)


def _final_body(accp_ref, g_ref, dinvx_ref, b_ref, wpbd_ref, fold_ref,
                wv_ref, bp_ref, bv_ref, proba_ref, value_ref):
    acc = accp_ref[0] + accp_ref[1] + g_ref[...]
    h3 = jnp.maximum(acc * dinvx_ref[...] + b_ref[...], 0.0)
    proba_ref[...] = jnp.dot(h3, wpbd_ref[...],
                             preferred_element_type=jnp.float32) + bp_ref[...]
    s128 = jnp.sum(h3[:NFR, :], axis=0, keepdims=True)
    feat = jnp.dot(s128, fold_ref[...],
                   preferred_element_type=jnp.float32) * (1.0 / N)
    value_ref[...] = jnp.dot(feat, wv_ref[...],
                             preferred_element_type=jnp.float32) + bv_ref[...]


def _tc_final(accpf, g3, dinvx, b3f, WpBD, FoldM, Wv, bp, bv):
    return pl.pallas_call(
        _final_body,
        out_shape=(
            jax.ShapeDtypeStruct((FR, 16), jnp.float32),
            jax.ShapeDtypeStruct((1, 1), jnp.float32),
        ),
    )(accpf, g3, dinvx, b3f, WpBD, FoldM, Wv, bp, bv)


def kernel(x, edge_index, W1, b1, W2, b2, W3, b3, Wp, bp, Wv, bv):
    edges = edge_index.astype(jnp.int32)
    f32 = jnp.float32

    ones_h = jnp.ones((C,), f32)
    z1 = jnp.zeros((ACC_N,), f32)
    z8 = jnp.zeros((ACC_N, H), f32)

    eye16 = jnp.eye(16, dtype=f32)
    xg = jnp.concatenate([x, jnp.zeros((ACC_N - N, D_IN), f32)]
                         ).reshape(FR, 16 * D_IN)
    BigW1 = jnp.kron(eye16, W1)
    Brd = jnp.kron(eye16, jnp.ones((1, H), f32))
    WBD2 = jnp.kron(eye16, W2)
    WBD3 = jnp.kron(eye16, W3)
    WpBD = jnp.kron(eye16, Wp)
    FoldM = jnp.kron(jnp.ones((16, 1), f32), jnp.eye(H, dtype=f32))

    degp = _sc_degree(edges, ones_h, z1)
    h1 = _tc_mm1(xg, BigW1)
    g1, dinvx = _tc_scale(h1, degp.reshape(NC, FR, 16), Brd)

    acc1 = _sc_scatter(g1.reshape(ACC_N, H), edges, z8)
    g2 = _tc_layer(acc1.reshape(NC, FR, 128), g1, dinvx, WBD2,
                   jnp.tile(b1, 16).reshape(1, 128))
    acc2 = _sc_scatter(g2.reshape(ACC_N, H), edges, z8)
    g3 = _tc_layer(acc2.reshape(NC, FR, 128), g2, dinvx, WBD3,
                   jnp.tile(b2, 16).reshape(1, 128))
    acc3 = _sc_scatter(g3.reshape(ACC_N, H), edges, z8)

    proba_f, value = _tc_final(acc3.reshape(NC, FR, 128), g3, dinvx,
                               jnp.tile(b3, 16).reshape(1, 128),
                               WpBD, FoldM, Wv,
                               bp.reshape(1, 1), bv.reshape(1, 1))
    proba = proba_f.reshape(ACC_N, 1)[:N]
    return (proba, value)
